# baseline (speedup 1.0000x reference)
import functools

import jax
import jax.numpy as jnp
from jax.experimental import pallas as pl
from jax.experimental.pallas import tpu as pltpu


_CHANNELS = ((1, 8), (8, 16), (16, 32), (32, 64), (64, 128))


def _dims(length):
    dims = []
    t = length
    for _ in _CHANNELS:
        t_conv = t - 2
        t_pool = t_conv // 3
        dims.append((t_conv, t_pool))
        t = t_pool
    return tuple(dims), t


def _body(stage_dims, t_gru, bb, length,
          x_ref,
          w1big, b1big, w2, b2, w3, b3, w4, b4, w5, b5,
          wiT, biT, whTb, bhTb,
          wl1T, bl1T, wl2T, bl2T,
          o_ref, h5_ref):
    f32 = jnp.float32

    h5v = h5_ref[...]
    planes5 = [h5v[:, t * bb:(t + 1) * bb] for t in range(t_gru)]
    h5rev = jnp.concatenate(planes5[::-1], axis=1)
    gi_f = jnp.dot(wiT[0], h5v, preferred_element_type=f32) + biT[0]
    gi_b = jnp.dot(wiT[1], h5rev, preferred_element_type=f32) + biT[1]

    h_both = jnp.zeros((64, bb), f32)
    h_max = None
    h_sum = None
    for t in range(t_gru):
        cs = slice(t * bb, (t + 1) * bb)
        gi = jnp.concatenate([gi_f[0:32, cs], gi_b[0:32, cs],
                              gi_f[32:64, cs], gi_b[32:64, cs],
                              gi_f[64:96, cs], gi_b[64:96, cs]], axis=0)
        gh = jnp.dot(whTb[...], h_both, preferred_element_type=f32) + bhTb[...]
        r = jax.nn.sigmoid(gi[0:64] + gh[0:64])
        z = jax.nn.sigmoid(gi[64:128] + gh[64:128])
        n = jnp.tanh(gi[128:192] + r * gh[128:192])
        h_both = (1.0 - z) * n + z * h_both
        h_max = h_both if h_max is None else jnp.maximum(h_max, h_both)
        h_sum = h_both if h_sum is None else h_sum + h_both

    feat = jnp.concatenate(
        [h_max, h_sum * (1.0 / t_gru), h_both], axis=0)
    hid = jnp.dot(wl1T[...], feat, preferred_element_type=f32) + bl1T[...]
    hid = jnp.maximum(hid, 0.0)
    logit = jnp.dot(wl2T[...], hid, preferred_element_type=f32) + bl2T[...]
    o_ref[...] = jax.nn.sigmoid(logit)

    tp1 = stage_dims[0][1]
    xall = jnp.transpose(x_ref[...].astype(jnp.bfloat16))
    nt = length // 8
    tiles = [xall[8 * t:8 * t + 8, :] for t in range(nt)]
    nj = (tp1 + 7) // 8
    wins = []
    for J in range(nj):
        idx = [min(3 * J + d, nt - 1) for d in range(4)]
        wins.append(jnp.concatenate([tiles[i] for i in idx], axis=0))
    x1 = jnp.concatenate(wins, axis=1)
    c = jnp.dot(w1big[...], x1, preferred_element_type=f32)
    p = jnp.maximum(jnp.maximum(c[0:64], c[64:128]), c[128:192])
    h = jnp.maximum(p + b1big[...], 0.0).astype(jnp.bfloat16)
    planes = [h[8 * (j % 8):8 * (j % 8) + 8,
               (j // 8) * bb:(j // 8 + 1) * bb] for j in range(tp1)]

    for s, (w_s, b_s) in enumerate(((w2, b2), (w3, b3), (w4, b4), (w5, b5)),
                                   start=1):
        tp = stage_dims[s][1]
        cout = _CHANNELS[s][1]
        xfull = jnp.concatenate(
            [jnp.concatenate([planes[3 * j + m] for j in range(tp)], axis=1)
             for m in range(5)], axis=0)
        c = jnp.dot(w_s[...], xfull, preferred_element_type=f32)
        p = jnp.maximum(jnp.maximum(c[0:cout], c[cout:2 * cout]),
                        c[2 * cout:3 * cout])
        h = jnp.maximum(p + b_s[...], 0.0)
        if s < 4:
            h = h.astype(jnp.bfloat16)
        planes = [h[:, j * bb:(j + 1) * bb] for j in range(tp)]

    h5_ref[...] = h


def kernel(x, c0w, c0b, c1w, c1b, c2w, c2b, c3w, c3b, c4w, c4b,
           gwi, gwh, gbi, gbh, mw1, mb1, mw2, mb2):
    batch, _, length = x.shape
    stage_dims, t_gru = _dims(length)

    bb = 512
    g = pl.cdiv(batch, bb)
    x2d = x.reshape(batch, length)
    if g * bb != batch:
        x2d = jnp.pad(x2d, ((0, g * bb - batch), (0, 0)))

    c0wT = jnp.transpose(c0w)
    w1big = jnp.concatenate([
        jnp.concatenate(
            [jnp.pad(c0wT, ((0, 0), (3 * u + r, 32 - 3 * u - r - 3)))
             for u in range(8)], axis=0)
        for r in range(3)])
    b1big = jnp.tile(jnp.transpose(c0b), (8, 1))
    conv_ws = [w1big.astype(jnp.bfloat16), b1big]
    for (cin, _), (w, b) in zip(_CHANNELS[1:],
                                ((c1w, c1b), (c2w, c2b), (c3w, c3b),
                                 (c4w, c4b))):
        wt = jnp.transpose(w)
        wstack = jnp.concatenate(
            [jnp.pad(wt, ((0, 0), (r * cin, (2 - r) * cin)))
             for r in range(3)], axis=0)
        conv_ws += [wstack.astype(jnp.bfloat16), jnp.transpose(b)]
    wiT = jnp.swapaxes(gwi, 1, 2)
    whT = jnp.swapaxes(gwh, 1, 2)
    biT = jnp.swapaxes(gbi, 1, 2)
    bhT = jnp.swapaxes(gbh, 1, 2)
    z3232 = jnp.zeros((32, 32), jnp.float32)
    wh_blocks = []
    bh_blocks = []
    for gs in (slice(0, 32), slice(32, 64), slice(64, 96)):
        wh_blocks.append(jnp.concatenate([whT[0][gs], z3232], axis=1))
        wh_blocks.append(jnp.concatenate([z3232, whT[1][gs]], axis=1))
        bh_blocks += [bhT[0][gs], bhT[1][gs]]
    whTb = jnp.concatenate(wh_blocks, axis=0)
    bhTb = jnp.concatenate(bh_blocks, axis=0)
    wl1T = jnp.transpose(mw1)
    bl1T = jnp.transpose(mb1)
    wl2T = jnp.transpose(mw2)
    bl2T = mb2
    weights = conv_ws + [wiT, biT, whTb, bhTb, wl1T, bl1T, wl2T, bl2T]

    def _const_spec(a):
        return pl.BlockSpec(a.shape, lambda i, _n=a.ndim: (0,) * _n)

    in_specs = [pl.BlockSpec((bb, length),
                             lambda i: (jnp.minimum(i, g - 1), 0))]
    in_specs += [_const_spec(a) for a in weights]

    scratch_shapes = [pltpu.VMEM((128, t_gru * bb), jnp.float32)]

    body = functools.partial(_body, stage_dims, t_gru, bb, length)
    out = pl.pallas_call(
        body,
        out_shape=jax.ShapeDtypeStruct((1, g * bb), jnp.float32),
        grid=(g + 1,),
        in_specs=in_specs,
        out_specs=pl.BlockSpec((1, bb), lambda i: (0, jnp.maximum(i - 1, 0))),
        scratch_shapes=scratch_shapes,
        compiler_params=pltpu.CompilerParams(
            dimension_semantics=("arbitrary",)),
    )(x2d, *weights)
    return out.reshape(g * bb, 1)[:batch]


# speedup vs baseline: 1.1478x; 1.1478x over previous
import functools

import jax
import jax.numpy as jnp
from jax.experimental import pallas as pl
from jax.experimental.pallas import tpu as pltpu


_CHANNELS = ((1, 8), (8, 16), (16, 32), (32, 64), (64, 128))


def _dims(length):
    dims = []
    t = length
    for _ in _CHANNELS:
        t_conv = t - 2
        t_pool = t_conv // 3
        dims.append((t_conv, t_pool))
        t = t_pool
    return tuple(dims), t


def _body(stage_dims, t_gru, bb, length,
          x_ref,
          w1big, b1big, w2, b2, w3, b3, w4, b4, w5, b5,
          wiT, biT, whTb, bhTb,
          wl1T, bl1T, wl2T, bl2T,
          o_ref, h5_ref):
    f32 = jnp.float32

    h5v = h5_ref[...]
    planes5 = [h5v[:, t * bb:(t + 1) * bb] for t in range(t_gru)]
    h5rev = jnp.concatenate(planes5[::-1], axis=1)
    gi_f = jnp.dot(wiT[0], h5v, preferred_element_type=f32) + biT[0]
    gi_b = jnp.dot(wiT[1], h5rev, preferred_element_type=f32) + biT[1]

    h_both = jnp.zeros((64, bb), f32)
    h_max = None
    h_sum = None
    for t in range(t_gru):
        cs = slice(t * bb, (t + 1) * bb)
        gi = jnp.concatenate([gi_f[0:32, cs], gi_b[0:32, cs],
                              gi_f[32:64, cs], gi_b[32:64, cs],
                              gi_f[64:96, cs], gi_b[64:96, cs]], axis=0)
        gh = jnp.dot(whTb[...], h_both, preferred_element_type=f32) + bhTb[...]
        r = jax.nn.sigmoid(gi[0:64] + gh[0:64])
        z = jax.nn.sigmoid(gi[64:128] + gh[64:128])
        n = jnp.tanh(gi[128:192] + r * gh[128:192])
        h_both = (1.0 - z) * n + z * h_both
        h_max = h_both if h_max is None else jnp.maximum(h_max, h_both)
        h_sum = h_both if h_sum is None else h_sum + h_both

    feat = jnp.concatenate(
        [h_max, h_sum * (1.0 / t_gru), h_both], axis=0)
    hid = jnp.dot(wl1T[...], feat, preferred_element_type=f32) + bl1T[...]
    hid = jnp.maximum(hid, 0.0)
    logit = jnp.dot(wl2T[...], hid, preferred_element_type=f32) + bl2T[...]
    o_ref[...] = jax.nn.sigmoid(logit)

    tp1 = stage_dims[0][1]
    xall = x_ref[...].astype(jnp.bfloat16)
    nt = length // 8
    tiles = [xall[8 * t:8 * t + 8, :] for t in range(nt)]
    nj = (tp1 + 7) // 8
    wins = []
    for J in range(nj):
        idx = [min(3 * J + d, nt - 1) for d in range(4)]
        wins.append(jnp.concatenate([tiles[i] for i in idx], axis=0))
    x1 = jnp.concatenate(wins, axis=1)
    c = jnp.dot(w1big[...], x1, preferred_element_type=f32)
    p = jnp.maximum(jnp.maximum(c[0:64], c[64:128]), c[128:192])
    h = jnp.maximum(p + b1big[...], 0.0).astype(jnp.bfloat16)
    planes = [h[8 * (j % 8):8 * (j % 8) + 8,
               (j // 8) * bb:(j // 8 + 1) * bb] for j in range(tp1)]

    for s, (w_s, b_s) in enumerate(((w2, b2), (w3, b3), (w4, b4), (w5, b5)),
                                   start=1):
        tp = stage_dims[s][1]
        cout = _CHANNELS[s][1]
        xfull = jnp.concatenate(
            [jnp.concatenate([planes[3 * j + m] for j in range(tp)], axis=1)
             for m in range(5)], axis=0)
        c = jnp.dot(w_s[...], xfull, preferred_element_type=f32)
        p = jnp.maximum(jnp.maximum(c[0:cout], c[cout:2 * cout]),
                        c[2 * cout:3 * cout])
        h = jnp.maximum(p + b_s[...], 0.0)
        if s < 4:
            h = h.astype(jnp.bfloat16)
        planes = [h[:, j * bb:(j + 1) * bb] for j in range(tp)]

    h5_ref[...] = h


def kernel(x, c0w, c0b, c1w, c1b, c2w, c2b, c3w, c3b, c4w, c4b,
           gwi, gwh, gbi, gbh, mw1, mb1, mw2, mb2):
    batch, _, length = x.shape
    stage_dims, t_gru = _dims(length)

    bb = 512
    g = pl.cdiv(batch, bb)
    xT = jnp.transpose(x.reshape(batch, length))
    if g * bb != batch:
        xT = jnp.pad(xT, ((0, 0), (0, g * bb - batch)))

    c0wT = jnp.transpose(c0w)
    w1big = jnp.concatenate([
        jnp.concatenate(
            [jnp.pad(c0wT, ((0, 0), (3 * u + r, 32 - 3 * u - r - 3)))
             for u in range(8)], axis=0)
        for r in range(3)])
    b1big = jnp.tile(jnp.transpose(c0b), (8, 1))
    conv_ws = [w1big.astype(jnp.bfloat16), b1big]
    for (cin, _), (w, b) in zip(_CHANNELS[1:],
                                ((c1w, c1b), (c2w, c2b), (c3w, c3b),
                                 (c4w, c4b))):
        wt = jnp.transpose(w)
        wstack = jnp.concatenate(
            [jnp.pad(wt, ((0, 0), (r * cin, (2 - r) * cin)))
             for r in range(3)], axis=0)
        conv_ws += [wstack.astype(jnp.bfloat16), jnp.transpose(b)]
    wiT = jnp.swapaxes(gwi, 1, 2)
    whT = jnp.swapaxes(gwh, 1, 2)
    biT = jnp.swapaxes(gbi, 1, 2)
    bhT = jnp.swapaxes(gbh, 1, 2)
    z3232 = jnp.zeros((32, 32), jnp.float32)
    wh_blocks = []
    bh_blocks = []
    for gs in (slice(0, 32), slice(32, 64), slice(64, 96)):
        wh_blocks.append(jnp.concatenate([whT[0][gs], z3232], axis=1))
        wh_blocks.append(jnp.concatenate([z3232, whT[1][gs]], axis=1))
        bh_blocks += [bhT[0][gs], bhT[1][gs]]
    whTb = jnp.concatenate(wh_blocks, axis=0)
    bhTb = jnp.concatenate(bh_blocks, axis=0)
    wl1T = jnp.transpose(mw1)
    bl1T = jnp.transpose(mb1)
    wl2T = jnp.transpose(mw2)
    bl2T = mb2
    weights = conv_ws + [wiT, biT, whTb, bhTb, wl1T, bl1T, wl2T, bl2T]

    def _const_spec(a):
        return pl.BlockSpec(a.shape, lambda i, _n=a.ndim: (0,) * _n)

    in_specs = [pl.BlockSpec((length, bb),
                             lambda i: (0, jnp.minimum(i, g - 1)))]
    in_specs += [_const_spec(a) for a in weights]

    scratch_shapes = [pltpu.VMEM((128, t_gru * bb), jnp.float32)]

    body = functools.partial(_body, stage_dims, t_gru, bb, length)
    out = pl.pallas_call(
        body,
        out_shape=jax.ShapeDtypeStruct((1, g * bb), jnp.float32),
        grid=(g + 1,),
        in_specs=in_specs,
        out_specs=pl.BlockSpec((1, bb), lambda i: (0, jnp.maximum(i - 1, 0))),
        scratch_shapes=scratch_shapes,
        compiler_params=pltpu.CompilerParams(
            dimension_semantics=("arbitrary",)),
    )(xT, *weights)
    return out.reshape(g * bb, 1)[:batch]
